# baseline (speedup 1.0000x reference)
import functools

import jax
import jax.numpy as jnp
from jax import lax
from jax.experimental import pallas as pl
from jax.experimental.pallas import tpu as pltpu
from jax.experimental.pallas import tpu_sc as plsc

_NC = 2
_NS = 16
_NW = _NC * _NS
_L = 16


def _sc_gather(me3, flat):
    nw, chunks, c = me3.shape
    _, d = flat.shape
    m = nw * chunks * c

    mesh = plsc.VectorSubcoreMesh(
        core_axis_name="c", subcore_axis_name="s",
        num_cores=_NC, num_subcores=_NS)

    @functools.partial(
        pl.kernel, mesh=mesh,
        out_type=jax.ShapeDtypeStruct((m, d), jnp.float32),
        scratch_types=[
            pltpu.VMEM((chunks, c), jnp.int32),
            pltpu.VMEM((2, c, d), jnp.float32),
            pltpu.SemaphoreType.DMA,
            pltpu.SemaphoreType.DMA,
            pltpu.SemaphoreType.DMA,
            pltpu.SemaphoreType.DMA,
        ],
    )
    def gather_k(me_hbm, flat_hbm, out_hbm, idx_v, rows2,
                 si0, si1, so0, so1):
        wid = lax.axis_index("s") * _NC + lax.axis_index("c")
        pltpu.sync_copy(me_hbm.at[wid], idx_v)
        base = wid * (chunks * c)
        sin = (si0, si1)
        sout = (so0, so1)
        in_d = [None] * chunks
        out_d = [None] * chunks

        def start_in(j):
            in_d[j] = pltpu.async_copy(
                flat_hbm.at[idx_v.at[j]], rows2.at[j % 2], sin[j % 2])

        def start_out(j):
            out_d[j] = pltpu.async_copy(
                rows2.at[j % 2], out_hbm.at[pl.ds(base + j * c, c)],
                sout[j % 2])

        start_in(0)
        for j in range(chunks):
            if j + 1 < chunks:
                if j >= 1:
                    out_d[j - 1].wait()
                start_in(j + 1)
            in_d[j].wait()
            start_out(j)
        for j in range(max(chunks - 2, 0), chunks):
            out_d[j].wait()

    return gather_k(me3, flat)


def _tc_mixer(gathered, w1b, w2c, bm=1024):
    m, d = gathered.shape
    _, h = w1b.shape

    def body(x_ref, w1_ref, w2_ref, o_ref):
        xb = x_ref[...].astype(jnp.bfloat16)
        acts = jnp.dot(xb, w1_ref[...], preferred_element_type=jnp.float32)
        acts = jax.nn.gelu(acts).astype(jnp.bfloat16)
        o_ref[...] = jnp.dot(acts, w2_ref[...],
                             preferred_element_type=jnp.float32)

    return pl.pallas_call(
        body,
        grid=(m // bm,),
        in_specs=[
            pl.BlockSpec((bm, d), lambda i: (i, 0)),
            pl.BlockSpec((d, h), lambda i: (0, 0)),
            pl.BlockSpec((h, 1), lambda i: (0, 0)),
        ],
        out_specs=pl.BlockSpec((bm, 1), lambda i: (i, 0)),
        out_shape=jax.ShapeDtypeStruct((m, 1), jnp.float32),
    )(gathered, w1b, w2c)


def _tc_oidx(me2, cu2, b, max_seqlen):
    rows, c = me2.shape

    def body(me_ref, cu_ref, o_ref):
        me = me_ref[...]
        doc = jnp.zeros(me.shape, jnp.int32)
        base = jnp.zeros(me.shape, jnp.int32)
        for j in range(1, b):
            cu_j = cu_ref[0, j]
            doc = doc + jnp.where(me >= cu_j, 1, 0)
        for j in range(1, b):
            base = base + jnp.where(doc == j, cu_ref[0, j], 0)
        o_ref[...] = doc * max_seqlen + me - base

    return pl.pallas_call(
        body,
        out_shape=jax.ShapeDtypeStruct((rows, c), jnp.int32),
    )(me2, cu2)


def _sc_scatter(pred_chunks, oidx_f, b, max_seqlen):
    (m,) = oidx_f.shape
    n_in = len(pred_chunks)
    sizes = [int(p.shape[0]) for p in pred_chunks]
    offs = [sum(sizes[:i]) for i in range(n_in)]
    per_tile = (b * max_seqlen) // _NS

    mesh = plsc.VectorSubcoreMesh(
        core_axis_name="c", subcore_axis_name="s",
        num_cores=1, num_subcores=_NS)

    @functools.partial(
        pl.kernel, mesh=mesh,
        compiler_params=pltpu.CompilerParams(needs_layout_passes=False),
        out_type=jax.ShapeDtypeStruct((b, max_seqlen), jnp.float32),
        scratch_types=[
            pltpu.VMEM((m,), jnp.float32),
            pltpu.VMEM((m,), jnp.int32),
            pltpu.VMEM((per_tile,), jnp.float32),
            pltpu.SemaphoreType.DMA,
        ],
    )
    def scatter_k(*refs):
        pred_hbms = refs[:n_in]
        oidx_hbm, out_hbm, pred_v, oidx_v, zbuf, sem = refs[n_in:]
        sid = lax.axis_index("s")
        cps = [
            pltpu.async_copy(pred_hbms[i],
                             pred_v.at[pl.ds(offs[i], sizes[i])], sem)
            for i in range(n_in)
        ]
        cps.append(pltpu.async_copy(oidx_hbm, oidx_v, sem))

        def zero_body(i, _):
            zbuf[pl.ds(i * _L, _L)] = jnp.zeros((_L,), jnp.float32)
            return 0
        lax.fori_loop(0, per_tile // _L, zero_body, 0)
        for cp in cps:
            cp.wait()

        lo = sid * per_tile
        inner = 8

        def scan_body(i, _):
            o = i * (inner * _L)
            for kk in range(inner):
                idx16 = oidx_v[pl.ds(o + kk * _L, _L)]
                p16 = pred_v[pl.ds(o + kk * _L, _L)]
                li = idx16 - lo
                msk = (li >= 0) & (li < per_tile)
                plsc.store_scatter(zbuf, [li], p16, mask=msk)
            return 0
        lax.fori_loop(0, m // (inner * _L), scan_body, 0)

        pltpu.sync_copy(zbuf, out_hbm.at[sid])

    return scatter_k(*pred_chunks, oidx_f)


def kernel(flat, cu_seqlens, match_ends, W1, w2):
    total_tok, d = flat.shape
    (m,) = match_ends.shape
    b = cu_seqlens.shape[0] - 1
    max_seqlen = 4096

    sizes = (m // 4, 3 * m // 8, 3 * m // 8)
    chunk = 16
    w1b = W1.astype(jnp.bfloat16)
    w2c = w2.reshape(d, 1).astype(jnp.bfloat16)
    pred_chunks = []
    off = 0
    for mc in sizes:
        me_i = lax.slice_in_dim(match_ends, off, off + mc)
        me3_i = me_i.reshape(_NW, mc // (_NW * chunk), chunk)
        g_i = _sc_gather(me3_i, flat)
        pred_chunks.append(_tc_mixer(g_i, w1b, w2c, bm=1024).reshape(mc))
        off += mc

    cu2 = jnp.concatenate(
        [cu_seqlens.astype(jnp.int32),
         jnp.zeros((1024 - cu_seqlens.shape[0],), jnp.int32)]).reshape(8, 128)
    oidx = _tc_oidx(match_ends.reshape(m // 128, 128), cu2, b, max_seqlen)
    return _sc_scatter(pred_chunks, oidx.reshape(m), b, max_seqlen)


# speedup vs baseline: 1.0279x; 1.0279x over previous
import functools

import jax
import jax.numpy as jnp
from jax import lax
from jax.experimental import pallas as pl
from jax.experimental.pallas import tpu as pltpu
from jax.experimental.pallas import tpu_sc as plsc

_NC = 2
_NS = 16
_NW = _NC * _NS
_L = 16


def _sc_gather(me3, flat):
    nw, chunks, c = me3.shape
    _, d = flat.shape
    m = nw * chunks * c

    mesh = plsc.VectorSubcoreMesh(
        core_axis_name="c", subcore_axis_name="s",
        num_cores=_NC, num_subcores=_NS)

    @functools.partial(
        pl.kernel, mesh=mesh,
        out_type=jax.ShapeDtypeStruct((m, d), jnp.float32),
        scratch_types=[
            pltpu.VMEM((chunks, c), jnp.int32),
            pltpu.VMEM((2, c, d), jnp.float32),
            pltpu.SemaphoreType.DMA,
            pltpu.SemaphoreType.DMA,
            pltpu.SemaphoreType.DMA,
            pltpu.SemaphoreType.DMA,
        ],
    )
    def gather_k(me_hbm, flat_hbm, out_hbm, idx_v, rows2,
                 si0, si1, so0, so1):
        wid = lax.axis_index("s") * _NC + lax.axis_index("c")
        pltpu.sync_copy(me_hbm.at[wid], idx_v)
        base = wid * (chunks * c)
        sin = (si0, si1)
        sout = (so0, so1)
        in_d = [None] * chunks
        out_d = [None] * chunks

        def start_in(j):
            in_d[j] = pltpu.async_copy(
                flat_hbm.at[idx_v.at[j]], rows2.at[j % 2], sin[j % 2])

        def start_out(j):
            out_d[j] = pltpu.async_copy(
                rows2.at[j % 2], out_hbm.at[pl.ds(base + j * c, c)],
                sout[j % 2])

        start_in(0)
        for j in range(chunks):
            if j + 1 < chunks:
                if j >= 1:
                    out_d[j - 1].wait()
                start_in(j + 1)
            in_d[j].wait()
            start_out(j)
        for j in range(max(chunks - 2, 0), chunks):
            out_d[j].wait()

    return gather_k(me3, flat)


def _tc_mixer(gathered, w1b, w2c, bm=1024):
    m, d = gathered.shape
    _, h = w1b.shape

    def body(x_ref, w1_ref, w2_ref, o_ref):
        xb = x_ref[...].astype(jnp.bfloat16)
        acts = jnp.dot(xb, w1_ref[...], preferred_element_type=jnp.float32)
        acts = jax.nn.gelu(acts).astype(jnp.bfloat16)
        o_ref[...] = jnp.dot(acts, w2_ref[...],
                             preferred_element_type=jnp.float32)

    return pl.pallas_call(
        body,
        grid=(m // bm,),
        in_specs=[
            pl.BlockSpec((bm, d), lambda i: (i, 0)),
            pl.BlockSpec((d, h), lambda i: (0, 0)),
            pl.BlockSpec((h, 1), lambda i: (0, 0)),
        ],
        out_specs=pl.BlockSpec((bm, 1), lambda i: (i, 0)),
        out_shape=jax.ShapeDtypeStruct((m, 1), jnp.float32),
    )(gathered, w1b, w2c)


def _tc_oidx(me2, cu2, b, max_seqlen):
    rows, c = me2.shape

    def body(me_ref, cu_ref, o_ref):
        me = me_ref[...]
        doc = jnp.zeros(me.shape, jnp.int32)
        base = jnp.zeros(me.shape, jnp.int32)
        for j in range(1, b):
            cu_j = cu_ref[0, j]
            doc = doc + jnp.where(me >= cu_j, 1, 0)
        for j in range(1, b):
            base = base + jnp.where(doc == j, cu_ref[0, j], 0)
        o_ref[...] = doc * max_seqlen + me - base

    return pl.pallas_call(
        body,
        out_shape=jax.ShapeDtypeStruct((rows, c), jnp.int32),
    )(me2, cu2)


def _sc_scatter(pred_chunks, oidx_f, b, max_seqlen):
    (m,) = oidx_f.shape
    n_in = len(pred_chunks)
    sizes = [int(p.shape[0]) for p in pred_chunks]
    offs = [sum(sizes[:i]) for i in range(n_in)]
    per_tile = (b * max_seqlen) // _NS

    mesh = plsc.VectorSubcoreMesh(
        core_axis_name="c", subcore_axis_name="s",
        num_cores=1, num_subcores=_NS)

    @functools.partial(
        pl.kernel, mesh=mesh,
        compiler_params=pltpu.CompilerParams(needs_layout_passes=False),
        out_type=jax.ShapeDtypeStruct((b, max_seqlen), jnp.float32),
        scratch_types=[
            pltpu.VMEM((m,), jnp.float32),
            pltpu.VMEM((m,), jnp.int32),
            pltpu.VMEM((per_tile,), jnp.float32),
            pltpu.SemaphoreType.DMA,
        ],
    )
    def scatter_k(*refs):
        pred_hbms = refs[:n_in]
        oidx_hbm, out_hbm, pred_v, oidx_v, zbuf, sem = refs[n_in:]
        sid = lax.axis_index("s")
        cps = [
            pltpu.async_copy(pred_hbms[i],
                             pred_v.at[pl.ds(offs[i], sizes[i])], sem)
            for i in range(n_in)
        ]
        cps.append(pltpu.async_copy(oidx_hbm, oidx_v, sem))

        def zero_body(i, _):
            zbuf[pl.ds(i * _L, _L)] = jnp.zeros((_L,), jnp.float32)
            return 0
        lax.fori_loop(0, per_tile // _L, zero_body, 0)
        for cp in cps:
            cp.wait()

        lo = sid * per_tile
        inner = 8

        def scan_body(i, _):
            o = i * (inner * _L)
            for kk in range(inner):
                idx16 = oidx_v[pl.ds(o + kk * _L, _L)]
                p16 = pred_v[pl.ds(o + kk * _L, _L)]
                li = idx16 - lo
                msk = (li >= 0) & (li < per_tile)
                plsc.store_scatter(zbuf, [li], p16, mask=msk)
            return 0
        lax.fori_loop(0, m // (inner * _L), scan_body, 0)

        pltpu.sync_copy(zbuf, out_hbm.at[sid])

    return scatter_k(*pred_chunks, oidx_f)


def kernel(flat, cu_seqlens, match_ends, W1, w2):
    total_tok, d = flat.shape
    (m,) = match_ends.shape
    b = cu_seqlens.shape[0] - 1
    max_seqlen = 4096

    sizes = (3 * m // 8, 5 * m // 8)
    chunk = 16
    w1b = W1.astype(jnp.bfloat16)
    w2c = w2.reshape(d, 1).astype(jnp.bfloat16)
    pred_chunks = []
    off = 0
    for mc in sizes:
        me_i = lax.slice_in_dim(match_ends, off, off + mc)
        me3_i = me_i.reshape(_NW, mc // (_NW * chunk), chunk)
        g_i = _sc_gather(me3_i, flat)
        pred_chunks.append(_tc_mixer(g_i, w1b, w2c, bm=1024).reshape(mc))
        off += mc

    cu2 = jnp.concatenate(
        [cu_seqlens.astype(jnp.int32),
         jnp.zeros((1024 - cu_seqlens.shape[0],), jnp.int32)]).reshape(8, 128)
    oidx = _tc_oidx(match_ends.reshape(m // 128, 128), cu2, b, max_seqlen)
    return _sc_scatter(pred_chunks, oidx.reshape(m), b, max_seqlen)


# speedup vs baseline: 1.0591x; 1.0303x over previous
import functools

import jax
import jax.numpy as jnp
from jax import lax
from jax.experimental import pallas as pl
from jax.experimental.pallas import tpu as pltpu
from jax.experimental.pallas import tpu_sc as plsc

_NC = 2
_NS = 16
_NW = _NC * _NS
_L = 16


def _sc_gather(me3, flat):
    nw, chunks, c = me3.shape
    _, d = flat.shape
    m = nw * chunks * c

    mesh = plsc.VectorSubcoreMesh(
        core_axis_name="c", subcore_axis_name="s",
        num_cores=_NC, num_subcores=_NS)

    @functools.partial(
        pl.kernel, mesh=mesh,
        out_type=jax.ShapeDtypeStruct((m, d), jnp.float32),
        scratch_types=[
            pltpu.VMEM((chunks, c), jnp.int32),
            pltpu.VMEM((3, c, d), jnp.float32),
            pltpu.SemaphoreType.DMA,
            pltpu.SemaphoreType.DMA,
            pltpu.SemaphoreType.DMA,
            pltpu.SemaphoreType.DMA,
            pltpu.SemaphoreType.DMA,
            pltpu.SemaphoreType.DMA,
        ],
    )
    def gather_k(me_hbm, flat_hbm, out_hbm, idx_v, rows3,
                 si0, si1, si2, so0, so1, so2):
        wid = lax.axis_index("s") * _NC + lax.axis_index("c")
        pltpu.sync_copy(me_hbm.at[wid], idx_v)
        base = wid * (chunks * c)
        sin = (si0, si1, si2)
        sout = (so0, so1, so2)
        nb = 3
        in_d = [None] * chunks
        out_d = [None] * chunks

        def start_in(j):
            in_d[j] = pltpu.async_copy(
                flat_hbm.at[idx_v.at[j]], rows3.at[j % nb], sin[j % nb])

        def start_out(j):
            out_d[j] = pltpu.async_copy(
                rows3.at[j % nb], out_hbm.at[pl.ds(base + j * c, c)],
                sout[j % nb])

        start_in(0)
        start_in(1)
        for j in range(chunks):
            if j + 2 < chunks:
                if j >= 1:
                    out_d[j - 1].wait()
                start_in(j + 2)
            in_d[j].wait()
            start_out(j)
        for j in range(max(chunks - 3, 0), chunks):
            out_d[j].wait()

    return gather_k(me3, flat)


def _tc_mixer(gathered, w1b, w2c, bm=1024):
    m, d = gathered.shape
    _, h = w1b.shape

    def body(x_ref, w1_ref, w2_ref, o_ref):
        xb = x_ref[...].astype(jnp.bfloat16)
        acts = jnp.dot(xb, w1_ref[...], preferred_element_type=jnp.float32)
        acts = jax.nn.gelu(acts).astype(jnp.bfloat16)
        o_ref[...] = jnp.dot(acts, w2_ref[...],
                             preferred_element_type=jnp.float32)

    return pl.pallas_call(
        body,
        grid=(m // bm,),
        in_specs=[
            pl.BlockSpec((bm, d), lambda i: (i, 0)),
            pl.BlockSpec((d, h), lambda i: (0, 0)),
            pl.BlockSpec((h, 1), lambda i: (0, 0)),
        ],
        out_specs=pl.BlockSpec((bm, 1), lambda i: (i, 0)),
        out_shape=jax.ShapeDtypeStruct((m, 1), jnp.float32),
    )(gathered, w1b, w2c)


def _tc_oidx(me2, cu2, b, max_seqlen):
    rows, c = me2.shape

    def body(me_ref, cu_ref, o_ref):
        me = me_ref[...]
        doc = jnp.zeros(me.shape, jnp.int32)
        base = jnp.zeros(me.shape, jnp.int32)
        for j in range(1, b):
            cu_j = cu_ref[0, j]
            doc = doc + jnp.where(me >= cu_j, 1, 0)
        for j in range(1, b):
            base = base + jnp.where(doc == j, cu_ref[0, j], 0)
        o_ref[...] = doc * max_seqlen + me - base

    return pl.pallas_call(
        body,
        out_shape=jax.ShapeDtypeStruct((rows, c), jnp.int32),
    )(me2, cu2)


def _sc_scatter(pred_chunks, oidx_f, b, max_seqlen):
    (m,) = oidx_f.shape
    n_in = len(pred_chunks)
    sizes = [int(p.shape[0]) for p in pred_chunks]
    offs = [sum(sizes[:i]) for i in range(n_in)]
    per_tile = (b * max_seqlen) // _NS

    mesh = plsc.VectorSubcoreMesh(
        core_axis_name="c", subcore_axis_name="s",
        num_cores=1, num_subcores=_NS)

    @functools.partial(
        pl.kernel, mesh=mesh,
        compiler_params=pltpu.CompilerParams(needs_layout_passes=False),
        out_type=jax.ShapeDtypeStruct((b, max_seqlen), jnp.float32),
        scratch_types=[
            pltpu.VMEM((m,), jnp.float32),
            pltpu.VMEM((m,), jnp.int32),
            pltpu.VMEM((per_tile,), jnp.float32),
            pltpu.SemaphoreType.DMA,
        ],
    )
    def scatter_k(*refs):
        pred_hbms = refs[:n_in]
        oidx_hbm, out_hbm, pred_v, oidx_v, zbuf, sem = refs[n_in:]
        sid = lax.axis_index("s")
        cps = [
            pltpu.async_copy(pred_hbms[i],
                             pred_v.at[pl.ds(offs[i], sizes[i])], sem)
            for i in range(n_in)
        ]
        cps.append(pltpu.async_copy(oidx_hbm, oidx_v, sem))

        def zero_body(i, _):
            zbuf[pl.ds(i * _L, _L)] = jnp.zeros((_L,), jnp.float32)
            return 0
        lax.fori_loop(0, per_tile // _L, zero_body, 0)
        for cp in cps:
            cp.wait()

        lo = sid * per_tile
        inner = 8

        def scan_body(i, _):
            o = i * (inner * _L)
            for kk in range(inner):
                idx16 = oidx_v[pl.ds(o + kk * _L, _L)]
                p16 = pred_v[pl.ds(o + kk * _L, _L)]
                li = idx16 - lo
                msk = (li >= 0) & (li < per_tile)
                plsc.store_scatter(zbuf, [li], p16, mask=msk)
            return 0
        lax.fori_loop(0, m // (inner * _L), scan_body, 0)

        pltpu.sync_copy(zbuf, out_hbm.at[sid])

    return scatter_k(*pred_chunks, oidx_f)


def kernel(flat, cu_seqlens, match_ends, W1, w2):
    total_tok, d = flat.shape
    (m,) = match_ends.shape
    b = cu_seqlens.shape[0] - 1
    max_seqlen = 4096

    sizes = (m // 2, m // 2)
    chunk = 16
    w1b = W1.astype(jnp.bfloat16)
    w2c = w2.reshape(d, 1).astype(jnp.bfloat16)
    pred_chunks = []
    off = 0
    for mc in sizes:
        me_i = lax.slice_in_dim(match_ends, off, off + mc)
        me3_i = me_i.reshape(_NW, mc // (_NW * chunk), chunk)
        g_i = _sc_gather(me3_i, flat)
        pred_chunks.append(_tc_mixer(g_i, w1b, w2c, bm=1024).reshape(mc))
        off += mc

    cu2 = jnp.concatenate(
        [cu_seqlens.astype(jnp.int32),
         jnp.zeros((1024 - cu_seqlens.shape[0],), jnp.int32)]).reshape(8, 128)
    oidx = _tc_oidx(match_ends.reshape(m // 128, 128), cu2, b, max_seqlen)
    return _sc_scatter(pred_chunks, oidx.reshape(m), b, max_seqlen)


# speedup vs baseline: 1.0820x; 1.0217x over previous
import functools

import jax
import jax.numpy as jnp
from jax import lax
from jax.experimental import pallas as pl
from jax.experimental.pallas import tpu as pltpu
from jax.experimental.pallas import tpu_sc as plsc

_NC = 2
_NS = 16
_NW = _NC * _NS
_L = 16


def _sc_gather(me3, flat):
    nw, chunks, c = me3.shape
    _, d = flat.shape
    m = nw * chunks * c

    mesh = plsc.VectorSubcoreMesh(
        core_axis_name="c", subcore_axis_name="s",
        num_cores=_NC, num_subcores=_NS)

    @functools.partial(
        pl.kernel, mesh=mesh,
        out_type=jax.ShapeDtypeStruct((m, d), jnp.float32),
        scratch_types=[
            pltpu.VMEM((chunks, c), jnp.int32),
            pltpu.VMEM((3, c, d), jnp.float32),
            pltpu.SemaphoreType.DMA,
            pltpu.SemaphoreType.DMA,
            pltpu.SemaphoreType.DMA,
            pltpu.SemaphoreType.DMA,
            pltpu.SemaphoreType.DMA,
            pltpu.SemaphoreType.DMA,
        ],
    )
    def gather_k(me_hbm, flat_hbm, out_hbm, idx_v, rows3,
                 si0, si1, si2, so0, so1, so2):
        wid = lax.axis_index("s") * _NC + lax.axis_index("c")
        pltpu.sync_copy(me_hbm.at[wid], idx_v)
        base = wid * (chunks * c)
        sin = (si0, si1, si2)
        sout = (so0, so1, so2)
        nb = 3
        in_d = [None] * chunks
        out_d = [None] * chunks

        def start_in(j):
            in_d[j] = pltpu.async_copy(
                flat_hbm.at[idx_v.at[j]], rows3.at[j % nb], sin[j % nb])

        def start_out(j):
            out_d[j] = pltpu.async_copy(
                rows3.at[j % nb], out_hbm.at[pl.ds(base + j * c, c)],
                sout[j % nb])

        start_in(0)
        start_in(1)
        for j in range(chunks):
            if j + 2 < chunks:
                if j >= 1:
                    out_d[j - 1].wait()
                start_in(j + 2)
            in_d[j].wait()
            start_out(j)
        for j in range(max(chunks - 3, 0), chunks):
            out_d[j].wait()

    return gather_k(me3, flat)


def _tc_mixer(gathered, w1b, w2c, bm=1024):
    m, d = gathered.shape
    _, h = w1b.shape

    def body(x_ref, w1_ref, w2_ref, o_ref):
        xb = x_ref[...].astype(jnp.bfloat16)
        acts = jnp.dot(xb, w1_ref[...], preferred_element_type=jnp.float32)
        acts = jax.nn.gelu(acts).astype(jnp.bfloat16)
        o_ref[...] = jnp.dot(acts, w2_ref[...],
                             preferred_element_type=jnp.float32)[:, 0]

    return pl.pallas_call(
        body,
        grid=(m // bm,),
        in_specs=[
            pl.BlockSpec((bm, d), lambda i: (i, 0)),
            pl.BlockSpec((d, h), lambda i: (0, 0)),
            pl.BlockSpec((h, 1), lambda i: (0, 0)),
        ],
        out_specs=pl.BlockSpec((bm,), lambda i: (i,)),
        out_shape=jax.ShapeDtypeStruct((m,), jnp.float32),
    )(gathered, w1b, w2c)


def _tc_oidx(me2, cu2, b, max_seqlen):
    rows, c = me2.shape

    def body(me_ref, cu_ref, o_ref):
        me = me_ref[...]
        doc = jnp.zeros(me.shape, jnp.int32)
        base = jnp.zeros(me.shape, jnp.int32)
        for j in range(1, b):
            cu_j = cu_ref[0, j]
            doc = doc + jnp.where(me >= cu_j, 1, 0)
        for j in range(1, b):
            base = base + jnp.where(doc == j, cu_ref[0, j], 0)
        o_ref[...] = doc * max_seqlen + me - base

    return pl.pallas_call(
        body,
        out_shape=jax.ShapeDtypeStruct((rows, c), jnp.int32),
    )(me2, cu2)


def _sc_scatter(pred_chunks, oidx_f, b, max_seqlen):
    (m,) = oidx_f.shape
    n_in = len(pred_chunks)
    sizes = [int(p.shape[0]) for p in pred_chunks]
    offs = [sum(sizes[:i]) for i in range(n_in)]
    per_tile = (b * max_seqlen) // _NS

    mesh = plsc.VectorSubcoreMesh(
        core_axis_name="c", subcore_axis_name="s",
        num_cores=1, num_subcores=_NS)

    @functools.partial(
        pl.kernel, mesh=mesh,
        compiler_params=pltpu.CompilerParams(needs_layout_passes=False),
        out_type=jax.ShapeDtypeStruct((b, max_seqlen), jnp.float32),
        scratch_types=[
            pltpu.VMEM((m,), jnp.float32),
            pltpu.VMEM((m,), jnp.int32),
            pltpu.VMEM((per_tile,), jnp.float32),
            pltpu.SemaphoreType.DMA,
        ],
    )
    def scatter_k(*refs):
        pred_hbms = refs[:n_in]
        oidx_hbm, out_hbm, pred_v, oidx_v, zbuf, sem = refs[n_in:]
        sid = lax.axis_index("s")
        cps = [
            pltpu.async_copy(pred_hbms[i],
                             pred_v.at[pl.ds(offs[i], sizes[i])], sem)
            for i in range(n_in)
        ]
        cps.append(pltpu.async_copy(oidx_hbm, oidx_v, sem))

        def zero_body(i, _):
            zbuf[pl.ds(i * _L, _L)] = jnp.zeros((_L,), jnp.float32)
            return 0
        lax.fori_loop(0, per_tile // _L, zero_body, 0)
        for cp in cps:
            cp.wait()

        lo = sid * per_tile
        inner = 8

        def scan_body(i, _):
            o = i * (inner * _L)
            for kk in range(inner):
                idx16 = oidx_v[pl.ds(o + kk * _L, _L)]
                p16 = pred_v[pl.ds(o + kk * _L, _L)]
                li = idx16 - lo
                msk = (li >= 0) & (li < per_tile)
                plsc.store_scatter(zbuf, [li], p16, mask=msk)
            return 0
        lax.fori_loop(0, m // (inner * _L), scan_body, 0)

        pltpu.sync_copy(zbuf, out_hbm.at[sid])

    return scatter_k(*pred_chunks, oidx_f)


def kernel(flat, cu_seqlens, match_ends, W1, w2):
    total_tok, d = flat.shape
    (m,) = match_ends.shape
    b = cu_seqlens.shape[0] - 1
    max_seqlen = 4096

    sizes = (m // 2, m // 2)
    chunk = 16
    w1b = W1.astype(jnp.bfloat16)
    w2c = w2.reshape(d, 1).astype(jnp.bfloat16)
    pred_chunks = []
    off = 0
    for mc in sizes:
        me_i = lax.slice_in_dim(match_ends, off, off + mc)
        me3_i = me_i.reshape(_NW, mc // (_NW * chunk), chunk)
        g_i = _sc_gather(me3_i, flat)
        pred_chunks.append(_tc_mixer(g_i, w1b, w2c, bm=1024))
        off += mc

    cu2 = jnp.concatenate(
        [cu_seqlens.astype(jnp.int32),
         jnp.zeros((1024 - cu_seqlens.shape[0],), jnp.int32)]).reshape(8, 128)
    oidx = _tc_oidx(match_ends.reshape(m // 128, 128), cu2, b, max_seqlen)
    return _sc_scatter(pred_chunks, oidx.reshape(m), b, max_seqlen)
